# baseline (speedup 1.0000x reference)
import functools

import jax
import jax.numpy as jnp
from jax.experimental import pallas as pl

jax.config.update('jax_default_matmul_precision', 'highest')

F32 = jnp.float32
K_CODES = 4096
D_CODE = 64
CHUNK = 512
N_CHUNKS = K_CODES // CHUNK
CC = 0.25
NB = 8


def _shift(v, d):
    if d == 0:
        return v
    t, c = v.shape
    z = jnp.zeros((abs(d), c), F32)
    if d > 0:
        return jnp.concatenate([v[d:], z], axis=0)
    return jnp.concatenate([z, v[:d]], axis=0)


def _mm(a, b):
    return jnp.dot(a, b, preferred_element_type=F32,
                   precision=jax.lax.Precision.HIGHEST)


def _conv_s1(x, wt, b):
    k = wt.shape[0]
    pad = (k - 1) // 2
    acc = jnp.broadcast_to(b, (x.shape[0], wt.shape[2])).astype(F32)
    for j in range(k):
        acc = acc + _mm(_shift(x, j - pad), wt[j])
    return acc


def _conv_s2(xe, xo, wt, b, pad):
    k = wt.shape[0]
    acc = jnp.broadcast_to(b, (xe.shape[0], wt.shape[2])).astype(F32)
    for j in range(k):
        r = j - pad
        part = xe if (r % 2 == 0) else xo
        acc = acc + _mm(_shift(part, r // 2), wt[j])
    return acc


def _conv_t2(x, wt, b):
    oe = b + _mm(x, wt[1]) + _mm(_shift(x, -1), wt[3])
    oo = b + _mm(_shift(x, 1), wt[0]) + _mm(x, wt[2])
    return oe, oo


def _resblock(x, w1, b1, w2, b2):
    h = jax.nn.relu(x)
    h = _conv_s1(h, w1, b1)
    h = jax.nn.relu(h)
    h = _mm(h, w2) + b2
    return x + h


def _enc1_body(x_ref, w1t_ref, b1_ref, r0w1_ref, r0b1_ref, r0w2_ref, r0b2_ref,
               r1w1_ref, r1b1_ref, r1w2_ref, r1b2_ref, o_ref):
    xeo = x_ref[0]
    w1t = w1t_ref[...]
    b1 = b1_ref[...]
    xe = xeo[:, 0:1]
    xo = xeo[:, 1:2]
    acc = jnp.broadcast_to(b1, (xe.shape[0], 128)).astype(F32)
    for j in range(7):
        r = j - 3
        part = xe if (r % 2 == 0) else xo
        acc = acc + _shift(part, r // 2) * w1t[j][None, :]
    h = _resblock(acc, r0w1_ref[...], r0b1_ref[...], r0w2_ref[...], r0b2_ref[...])
    h = _resblock(h, r1w1_ref[...], r1b1_ref[...], r1w2_ref[...], r1b2_ref[...])
    o_ref[0] = h


def _enc2_body(xe_ref, xo_ref, wt_ref, b_ref, r0w1_ref, r0b1_ref, r0w2_ref,
               r0b2_ref, r1w1_ref, r1b1_ref, r1w2_ref, r1b2_ref, o_ref, *, pad):
    h = _conv_s2(xe_ref[0], xo_ref[0], wt_ref[...], b_ref[...], pad)
    h = _resblock(h, r0w1_ref[...], r0b1_ref[...], r0w2_ref[...], r0b2_ref[...])
    h = _resblock(h, r1w1_ref[...], r1b1_ref[...], r1w2_ref[...], r1b2_ref[...])
    o_ref[0] = h


def _enc3_body(xe_ref, xo_ref, wt_ref, b_ref, r0w1_ref, r0b1_ref, r0w2_ref,
               r0b2_ref, r1w1_ref, r1b1_ref, r1w2_ref, r1b2_ref,
               pw_ref, pb_ref, o_ref, *, pad):
    h = _conv_s2(xe_ref[0], xo_ref[0], wt_ref[...], b_ref[...], pad)
    h = _resblock(h, r0w1_ref[...], r0b1_ref[...], r0w2_ref[...], r0b2_ref[...])
    h = _resblock(h, r1w1_ref[...], r1b1_ref[...], r1w2_ref[...], r1b2_ref[...])
    o_ref[0] = _mm(h, pw_ref[...]) + pb_ref[...]


def _quantize_one(xf, embT, emb):
    t = xf.shape[0]
    x2 = jnp.sum(xf * xf, axis=1, keepdims=True)
    best_d = jnp.full((t, 1), jnp.inf, F32)
    best_i = jnp.zeros((t, 1), jnp.int32)
    for c in range(N_CHUNKS):
        et = embT[:, c * CHUNK:(c + 1) * CHUNK]
        d = x2 - 2.0 * _mm(xf, et) + jnp.sum(et * et, axis=0, keepdims=True)
        m = jnp.min(d, axis=1, keepdims=True)
        iota = jax.lax.broadcasted_iota(jnp.int32, (t, CHUNK), 1)
        ii = jnp.min(jnp.where(d <= m, iota, K_CODES), axis=1, keepdims=True)
        upd = m < best_d
        best_d = jnp.where(upd, m, best_d)
        best_i = jnp.where(upd, ii + c * CHUNK, best_i)
    quant = jnp.zeros((t, D_CODE), F32)
    counts = []
    for c in range(N_CHUNKS):
        iota = jax.lax.broadcasted_iota(jnp.int32, (t, CHUNK), 1) + c * CHUNK
        oh = (best_i == iota).astype(F32)
        quant = quant + _mm(oh, emb[c * CHUNK:(c + 1) * CHUNK])
        counts.append(jnp.sum(oh, axis=0, keepdims=True))
    counts = jnp.concatenate(counts, axis=1)
    sumsq = jnp.sum((quant - xf) ** 2)
    return quant, counts, sumsq


def _quant_body(z_ref, e1t_ref, e1_ref, e2t_ref, e2_ref,
                q_ref, c1_ref, c2_ref, commit_ref, p1_ref, p2_ref, u_ref):
    b = pl.program_id(0)
    z = z_ref[0]
    q1, cnt1, s1 = _quantize_one(z[:, :D_CODE], e1t_ref[...], e1_ref[...])
    q2, cnt2, s2 = _quantize_one(z[:, D_CODE:], e2t_ref[...], e2_ref[...])
    q_ref[0] = jnp.concatenate([q1, q2], axis=1)

    @pl.when(b == 0)
    def _init():
        c1_ref[...] = cnt1
        c2_ref[...] = cnt2
        commit_ref[...] = jnp.full((1, 1), s1 + s2, F32)

    @pl.when(b > 0)
    def _acc():
        c1_ref[...] += cnt1
        c2_ref[...] += cnt2
        commit_ref[...] += s1 + s2

    @pl.when(b == NB - 1)
    def _finalize():
        n_rows = F32(NB * 1024)
        commit_ref[...] = commit_ref[...] * (CC / (NB * 1024 * D_CODE))
        for cref, pref in ((c1_ref, p1_ref), (c2_ref, p2_ref)):
            p = cref[...] / n_rows
            perp = jnp.exp(-jnp.sum(p * jnp.log(p + 1e-10)))
            pref[...] = jnp.full((1, 1), perp, F32)
        u1 = jnp.sum((c1_ref[...] / n_rows) *
                     jnp.log(c1_ref[...] / n_rows * K_CODES + 1e-10))
        u2 = jnp.sum((c2_ref[...] / n_rows) *
                     jnp.log(c2_ref[...] / n_rows * K_CODES + 1e-10))
        u_ref[...] = jnp.full((1, 1), u1 + u2, F32)


def _dec1_body(q_ref, pw_ref, pb_ref, r0w1_ref, r0b1_ref, r0w2_ref, r0b2_ref,
               r1w1_ref, r1b1_ref, r1w2_ref, r1b2_ref, uw_ref, ub_ref,
               oe_ref, oo_ref):
    h = _mm(q_ref[0], pw_ref[...]) + pb_ref[...]
    h = _resblock(h, r0w1_ref[...], r0b1_ref[...], r0w2_ref[...], r0b2_ref[...])
    h = _resblock(h, r1w1_ref[...], r1b1_ref[...], r1w2_ref[...], r1b2_ref[...])
    oe, oo = _conv_t2(h, uw_ref[...], ub_ref[...])
    oe_ref[0] = oe
    oo_ref[0] = oo


def _dec23_body(y_ref, r0w1_ref, r0b1_ref, r0w2_ref, r0b2_ref,
                r1w1_ref, r1b1_ref, r1w2_ref, r1b2_ref, uw_ref, ub_ref,
                oe_ref, oo_ref):
    h = y_ref[0]
    h = _resblock(h, r0w1_ref[...], r0b1_ref[...], r0w2_ref[...], r0b2_ref[...])
    h = _resblock(h, r1w1_ref[...], r1b1_ref[...], r1w2_ref[...], r1b2_ref[...])
    oe, oo = _conv_t2(h, uw_ref[...], ub_ref[...])
    oe_ref[0] = oe
    oo_ref[0] = oo


def _out_body(y_ref, wt_ref, b_ref, o_ref):
    y = y_ref[0]
    wt = wt_ref[...]
    acc = jnp.zeros((y.shape[0], 1), F32)
    for j in range(7):
        acc = acc + jnp.sum(_shift(y, j - 3) * wt[j][None, :], axis=1,
                            keepdims=True)
    o_ref[0] = acc + b_ref[0, 0]


def _bspec(shape):
    nd = len(shape)
    return pl.BlockSpec((1,) + tuple(shape[1:]),
                        lambda b: (b,) + (0,) * (nd - 1))


def _wspec(shape):
    nd = len(shape)
    return pl.BlockSpec(tuple(shape), lambda b: (0,) * nd)


def _call(body, inputs, batch_in_mask, out_shapes, batch_out_mask):
    in_specs = [_bspec(a.shape) if m else _wspec(a.shape)
                for a, m in zip(inputs, batch_in_mask)]
    out_specs = [_bspec(s) if m else _wspec(s)
                 for s, m in zip(out_shapes, batch_out_mask)]
    out_shape = [jax.ShapeDtypeStruct(s, F32) for s in out_shapes]
    if len(out_shapes) == 1:
        out_specs = out_specs[0]
        out_shape = out_shape[0]
    return pl.pallas_call(
        body,
        grid=(NB,),
        in_specs=in_specs,
        out_specs=out_specs,
        out_shape=out_shape,
    )(*inputs)


def _kt(w):
    return jnp.transpose(w, (2, 1, 0))


def _ktT(w):
    return jnp.transpose(w, (2, 0, 1))


def _rb(p, prefix, i):
    return (_kt(p['%s_%d_w1' % (prefix, i)]),
            p['%s_%d_b1' % (prefix, i)][None, :],
            p['%s_%d_w2' % (prefix, i)][:, :, 0].T,
            p['%s_%d_b2' % (prefix, i)][None, :])


def kernel(x, params):
    p = params
    xeo = jnp.reshape(x[:, 0, :], (NB, 4096, 2))

    h1 = _call(
        _enc1_body,
        [xeo, jnp.transpose(p['conv1_w'][:, 0, :]), p['conv1_b'][None, :],
         *_rb(p, 'enc_res1', 0), *_rb(p, 'enc_res1', 1)],
        [True] + [False] * 10,
        [(NB, 4096, 128)], [True])

    h1r = jnp.reshape(h1, (NB, 2048, 2, 128))
    h2 = _call(
        functools.partial(_enc2_body, pad=2),
        [h1r[:, :, 0, :], h1r[:, :, 1, :], _kt(p['conv2_w']),
         p['conv2_b'][None, :], *_rb(p, 'enc_res2', 0), *_rb(p, 'enc_res2', 1)],
        [True, True] + [False] * 10,
        [(NB, 2048, 256)], [True])

    h2r = jnp.reshape(h2, (NB, 1024, 2, 256))
    z = _call(
        functools.partial(_enc3_body, pad=1),
        [h2r[:, :, 0, :], h2r[:, :, 1, :], _kt(p['conv3_w']),
         p['conv3_b'][None, :], *_rb(p, 'enc_res3', 0), *_rb(p, 'enc_res3', 1),
         p['enc_proj_w'][:, :, 0].T, p['enc_proj_b'][None, :]],
        [True, True] + [False] * 12,
        [(NB, 1024, 128)], [True])

    q, _c1, _c2, commit, perp1, perp2, usage = _call(
        _quant_body,
        [z, p['embed1'].T, p['embed1'], p['embed2'].T, p['embed2']],
        [True] + [False] * 4,
        [(NB, 1024, 128), (1, K_CODES), (1, K_CODES), (1, 1), (1, 1), (1, 1),
         (1, 1)],
        [True, False, False, False, False, False, False])

    ye, yo = _call(
        _dec1_body,
        [q, p['dec_proj_w'][:, :, 0].T, p['dec_proj_b'][None, :],
         *_rb(p, 'dec_res1', 0), *_rb(p, 'dec_res1', 1),
         _ktT(p['up1_w']), p['up1_b'][None, :]],
        [True] + [False] * 12,
        [(NB, 1024, 256), (NB, 1024, 256)], [True, True])
    y = jnp.reshape(jnp.stack([ye, yo], axis=2), (NB, 2048, 256))

    ye, yo = _call(
        _dec23_body,
        [y, *_rb(p, 'dec_res2', 0), *_rb(p, 'dec_res2', 1),
         _ktT(p['up2_w']), p['up2_b'][None, :]],
        [True] + [False] * 10,
        [(NB, 2048, 128), (NB, 2048, 128)], [True, True])
    y = jnp.reshape(jnp.stack([ye, yo], axis=2), (NB, 4096, 128))

    ye, yo = _call(
        _dec23_body,
        [y, *_rb(p, 'dec_res3', 0), *_rb(p, 'dec_res3', 1),
         _ktT(p['up3_w']), p['up3_b'][None, :]],
        [True] + [False] * 10,
        [(NB, 4096, 128), (NB, 4096, 128)], [True, True])
    y = jnp.reshape(jnp.stack([ye, yo], axis=2), (NB, 8192, 128))

    recon = _call(
        _out_body,
        [y, jnp.transpose(p['out_w'][0]), p['out_b'][None, :]],
        [True, False, False],
        [(NB, 8192, 1)], [True])

    recon = jnp.transpose(recon, (0, 2, 1))
    return (recon, commit[0, 0], perp1[0, 0], perp2[0, 0], usage[0, 0])


# speedup vs baseline: 1.3371x; 1.3371x over previous
import functools

import jax
import jax.numpy as jnp
from jax.experimental import pallas as pl

jax.config.update('jax_default_matmul_precision', 'highest')

F32 = jnp.float32
K_CODES = 4096
D_CODE = 64
CHUNK = 512
N_CHUNKS = K_CODES // CHUNK
CC = 0.25
NB = 8


def _shift(v, d):
    if d == 0:
        return v
    t, c = v.shape
    z = jnp.zeros((abs(d), c), F32)
    if d > 0:
        return jnp.concatenate([v[d:], z], axis=0)
    return jnp.concatenate([z, v[:d]], axis=0)


def _mm(a, b, hi=True):
    prec = jax.lax.Precision.HIGHEST if hi else jax.lax.Precision.DEFAULT
    return jnp.dot(a, b, preferred_element_type=F32, precision=prec)


def _conv_s1(x, wt, b, hi=True):
    k = wt.shape[0]
    pad = (k - 1) // 2
    acc = jnp.broadcast_to(b, (x.shape[0], wt.shape[2])).astype(F32)
    for j in range(k):
        acc = acc + _mm(_shift(x, j - pad), wt[j], hi)
    return acc


def _conv_s2(xe, xo, wt, b, pad):
    k = wt.shape[0]
    acc = jnp.broadcast_to(b, (xe.shape[0], wt.shape[2])).astype(F32)
    for j in range(k):
        r = j - pad
        part = xe if (r % 2 == 0) else xo
        acc = acc + _mm(_shift(part, r // 2), wt[j])
    return acc


def _conv_t2(x, wt, b, hi=True):
    oe = b + _mm(x, wt[1], hi) + _mm(_shift(x, -1), wt[3], hi)
    oo = b + _mm(_shift(x, 1), wt[0], hi) + _mm(x, wt[2], hi)
    return oe, oo


def _resblock(x, w1, b1, w2, b2, hi=True):
    h = jax.nn.relu(x)
    h = _conv_s1(h, w1, b1, hi)
    h = jax.nn.relu(h)
    h = _mm(h, w2, hi) + b2
    return x + h


def _enc1_body(x_ref, w1t_ref, b1_ref, r0w1_ref, r0b1_ref, r0w2_ref, r0b2_ref,
               r1w1_ref, r1b1_ref, r1w2_ref, r1b2_ref, o_ref):
    xeo = x_ref[0]
    w1t = w1t_ref[...]
    b1 = b1_ref[...]
    xe = xeo[:, 0:1]
    xo = xeo[:, 1:2]
    acc = jnp.broadcast_to(b1, (xe.shape[0], 128)).astype(F32)
    for j in range(7):
        r = j - 3
        part = xe if (r % 2 == 0) else xo
        acc = acc + _shift(part, r // 2) * w1t[j][None, :]
    h = _resblock(acc, r0w1_ref[...], r0b1_ref[...], r0w2_ref[...], r0b2_ref[...])
    h = _resblock(h, r1w1_ref[...], r1b1_ref[...], r1w2_ref[...], r1b2_ref[...])
    o_ref[0] = h


def _enc2_body(xe_ref, xo_ref, wt_ref, b_ref, r0w1_ref, r0b1_ref, r0w2_ref,
               r0b2_ref, r1w1_ref, r1b1_ref, r1w2_ref, r1b2_ref, o_ref, *, pad):
    h = _conv_s2(xe_ref[0], xo_ref[0], wt_ref[...], b_ref[...], pad)
    h = _resblock(h, r0w1_ref[...], r0b1_ref[...], r0w2_ref[...], r0b2_ref[...])
    h = _resblock(h, r1w1_ref[...], r1b1_ref[...], r1w2_ref[...], r1b2_ref[...])
    o_ref[0] = h


def _enc3_body(xe_ref, xo_ref, wt_ref, b_ref, r0w1_ref, r0b1_ref, r0w2_ref,
               r0b2_ref, r1w1_ref, r1b1_ref, r1w2_ref, r1b2_ref,
               pw_ref, pb_ref, o_ref, *, pad):
    h = _conv_s2(xe_ref[0], xo_ref[0], wt_ref[...], b_ref[...], pad)
    h = _resblock(h, r0w1_ref[...], r0b1_ref[...], r0w2_ref[...], r0b2_ref[...])
    h = _resblock(h, r1w1_ref[...], r1b1_ref[...], r1w2_ref[...], r1b2_ref[...])
    o_ref[0] = _mm(h, pw_ref[...]) + pb_ref[...]


def _quantize_one(xf, embT, emb):
    t = xf.shape[0]
    x2 = jnp.sum(xf * xf, axis=1, keepdims=True)
    best_d = jnp.full((t, 1), jnp.inf, F32)
    best_i = jnp.zeros((t, 1), jnp.int32)
    for c in range(N_CHUNKS):
        et = embT[:, c * CHUNK:(c + 1) * CHUNK]
        d = x2 - 2.0 * _mm(xf, et) + jnp.sum(et * et, axis=0, keepdims=True)
        m = jnp.min(d, axis=1, keepdims=True)
        iota = jax.lax.broadcasted_iota(jnp.int32, (t, CHUNK), 1)
        ii = jnp.min(jnp.where(d <= m, iota, K_CODES), axis=1, keepdims=True)
        upd = m < best_d
        best_d = jnp.where(upd, m, best_d)
        best_i = jnp.where(upd, ii + c * CHUNK, best_i)
    quant = jnp.zeros((t, D_CODE), F32)
    counts = []
    for c in range(N_CHUNKS):
        iota = jax.lax.broadcasted_iota(jnp.int32, (t, CHUNK), 1) + c * CHUNK
        oh = (best_i == iota).astype(F32)
        quant = quant + _mm(oh, emb[c * CHUNK:(c + 1) * CHUNK])
        counts.append(jnp.sum(oh, axis=0, keepdims=True))
    counts = jnp.concatenate(counts, axis=1)
    sumsq = jnp.sum((quant - xf) ** 2)
    return quant, counts, sumsq


def _quant_body(z_ref, e1t_ref, e1_ref, e2t_ref, e2_ref,
                q_ref, c1_ref, c2_ref, commit_ref, p1_ref, p2_ref, u_ref):
    b = pl.program_id(0)
    z = z_ref[0]
    q1, cnt1, s1 = _quantize_one(z[:, :D_CODE], e1t_ref[...], e1_ref[...])
    q2, cnt2, s2 = _quantize_one(z[:, D_CODE:], e2t_ref[...], e2_ref[...])
    q_ref[0] = jnp.concatenate([q1, q2], axis=1)

    @pl.when(b == 0)
    def _init():
        c1_ref[...] = cnt1
        c2_ref[...] = cnt2
        commit_ref[...] = jnp.full((1, 1), s1 + s2, F32)

    @pl.when(b > 0)
    def _acc():
        c1_ref[...] += cnt1
        c2_ref[...] += cnt2
        commit_ref[...] += s1 + s2

    @pl.when(b == NB - 1)
    def _finalize():
        n_rows = F32(NB * 1024)
        commit_ref[...] = commit_ref[...] * (CC / (NB * 1024 * D_CODE))
        for cref, pref in ((c1_ref, p1_ref), (c2_ref, p2_ref)):
            p = cref[...] / n_rows
            perp = jnp.exp(-jnp.sum(p * jnp.log(p + 1e-10)))
            pref[...] = jnp.full((1, 1), perp, F32)
        u1 = jnp.sum((c1_ref[...] / n_rows) *
                     jnp.log(c1_ref[...] / n_rows * K_CODES + 1e-10))
        u2 = jnp.sum((c2_ref[...] / n_rows) *
                     jnp.log(c2_ref[...] / n_rows * K_CODES + 1e-10))
        u_ref[...] = jnp.full((1, 1), u1 + u2, F32)


def _dec1_body(q_ref, pw_ref, pb_ref, r0w1_ref, r0b1_ref, r0w2_ref, r0b2_ref,
               r1w1_ref, r1b1_ref, r1w2_ref, r1b2_ref, uw_ref, ub_ref,
               oe_ref, oo_ref):
    h = _mm(q_ref[0], pw_ref[...], False) + pb_ref[...]
    h = _resblock(h, r0w1_ref[...], r0b1_ref[...], r0w2_ref[...], r0b2_ref[...],
                  False)
    h = _resblock(h, r1w1_ref[...], r1b1_ref[...], r1w2_ref[...], r1b2_ref[...],
                  False)
    oe, oo = _conv_t2(h, uw_ref[...], ub_ref[...], False)
    oe_ref[0] = oe
    oo_ref[0] = oo


def _dec23_body(y_ref, r0w1_ref, r0b1_ref, r0w2_ref, r0b2_ref,
                r1w1_ref, r1b1_ref, r1w2_ref, r1b2_ref, uw_ref, ub_ref,
                oe_ref, oo_ref):
    h = y_ref[0]
    h = _resblock(h, r0w1_ref[...], r0b1_ref[...], r0w2_ref[...], r0b2_ref[...],
                  False)
    h = _resblock(h, r1w1_ref[...], r1b1_ref[...], r1w2_ref[...], r1b2_ref[...],
                  False)
    oe, oo = _conv_t2(h, uw_ref[...], ub_ref[...], False)
    oe_ref[0] = oe
    oo_ref[0] = oo


def _out_body(y_ref, wt_ref, b_ref, o_ref):
    y = y_ref[0]
    wt = wt_ref[...]
    acc = jnp.zeros((y.shape[0], 1), F32)
    for j in range(7):
        acc = acc + jnp.sum(_shift(y, j - 3) * wt[j][None, :], axis=1,
                            keepdims=True)
    o_ref[0] = acc + b_ref[0, 0]


def _bspec(shape):
    nd = len(shape)
    return pl.BlockSpec((1,) + tuple(shape[1:]),
                        lambda b: (b,) + (0,) * (nd - 1))


def _wspec(shape):
    nd = len(shape)
    return pl.BlockSpec(tuple(shape), lambda b: (0,) * nd)


def _call(body, inputs, batch_in_mask, out_shapes, batch_out_mask):
    in_specs = [_bspec(a.shape) if m else _wspec(a.shape)
                for a, m in zip(inputs, batch_in_mask)]
    out_specs = [_bspec(s) if m else _wspec(s)
                 for s, m in zip(out_shapes, batch_out_mask)]
    out_shape = [jax.ShapeDtypeStruct(s, F32) for s in out_shapes]
    if len(out_shapes) == 1:
        out_specs = out_specs[0]
        out_shape = out_shape[0]
    return pl.pallas_call(
        body,
        grid=(NB,),
        in_specs=in_specs,
        out_specs=out_specs,
        out_shape=out_shape,
    )(*inputs)


def _kt(w):
    return jnp.transpose(w, (2, 1, 0))


def _ktT(w):
    return jnp.transpose(w, (2, 0, 1))


def _rb(p, prefix, i):
    return (_kt(p['%s_%d_w1' % (prefix, i)]),
            p['%s_%d_b1' % (prefix, i)][None, :],
            p['%s_%d_w2' % (prefix, i)][:, :, 0].T,
            p['%s_%d_b2' % (prefix, i)][None, :])


def kernel(x, params):
    p = params
    xeo = jnp.reshape(x[:, 0, :], (NB, 4096, 2))

    h1 = _call(
        _enc1_body,
        [xeo, jnp.transpose(p['conv1_w'][:, 0, :]), p['conv1_b'][None, :],
         *_rb(p, 'enc_res1', 0), *_rb(p, 'enc_res1', 1)],
        [True] + [False] * 10,
        [(NB, 4096, 128)], [True])

    h1r = jnp.reshape(h1, (NB, 2048, 2, 128))
    h2 = _call(
        functools.partial(_enc2_body, pad=2),
        [h1r[:, :, 0, :], h1r[:, :, 1, :], _kt(p['conv2_w']),
         p['conv2_b'][None, :], *_rb(p, 'enc_res2', 0), *_rb(p, 'enc_res2', 1)],
        [True, True] + [False] * 10,
        [(NB, 2048, 256)], [True])

    h2r = jnp.reshape(h2, (NB, 1024, 2, 256))
    z = _call(
        functools.partial(_enc3_body, pad=1),
        [h2r[:, :, 0, :], h2r[:, :, 1, :], _kt(p['conv3_w']),
         p['conv3_b'][None, :], *_rb(p, 'enc_res3', 0), *_rb(p, 'enc_res3', 1),
         p['enc_proj_w'][:, :, 0].T, p['enc_proj_b'][None, :]],
        [True, True] + [False] * 12,
        [(NB, 1024, 128)], [True])

    q, _c1, _c2, commit, perp1, perp2, usage = _call(
        _quant_body,
        [z, p['embed1'].T, p['embed1'], p['embed2'].T, p['embed2']],
        [True] + [False] * 4,
        [(NB, 1024, 128), (1, K_CODES), (1, K_CODES), (1, 1), (1, 1), (1, 1),
         (1, 1)],
        [True, False, False, False, False, False, False])

    ye, yo = _call(
        _dec1_body,
        [q, p['dec_proj_w'][:, :, 0].T, p['dec_proj_b'][None, :],
         *_rb(p, 'dec_res1', 0), *_rb(p, 'dec_res1', 1),
         _ktT(p['up1_w']), p['up1_b'][None, :]],
        [True] + [False] * 12,
        [(NB, 1024, 256), (NB, 1024, 256)], [True, True])
    y = jnp.reshape(jnp.stack([ye, yo], axis=2), (NB, 2048, 256))

    ye, yo = _call(
        _dec23_body,
        [y, *_rb(p, 'dec_res2', 0), *_rb(p, 'dec_res2', 1),
         _ktT(p['up2_w']), p['up2_b'][None, :]],
        [True] + [False] * 10,
        [(NB, 2048, 128), (NB, 2048, 128)], [True, True])
    y = jnp.reshape(jnp.stack([ye, yo], axis=2), (NB, 4096, 128))

    ye, yo = _call(
        _dec23_body,
        [y, *_rb(p, 'dec_res3', 0), *_rb(p, 'dec_res3', 1),
         _ktT(p['up3_w']), p['up3_b'][None, :]],
        [True] + [False] * 10,
        [(NB, 4096, 128), (NB, 4096, 128)], [True, True])
    y = jnp.reshape(jnp.stack([ye, yo], axis=2), (NB, 8192, 128))

    recon = _call(
        _out_body,
        [y, jnp.transpose(p['out_w'][0]), p['out_b'][None, :]],
        [True, False, False],
        [(NB, 8192, 1)], [True])

    recon = jnp.transpose(recon, (0, 2, 1))
    return (recon, commit[0, 0], perp1[0, 0], perp2[0, 0], usage[0, 0])


# speedup vs baseline: 1.5110x; 1.1301x over previous
import functools

import jax
import jax.numpy as jnp
from jax.experimental import pallas as pl

jax.config.update('jax_default_matmul_precision', 'highest')

F32 = jnp.float32
K_CODES = 4096
D_CODE = 64
CHUNK = 512
N_CHUNKS = K_CODES // CHUNK
CC = 0.25
NB = 8


def _shift(v, d):
    if d == 0:
        return v
    t, c = v.shape
    z = jnp.zeros((abs(d), c), F32)
    if d > 0:
        return jnp.concatenate([v[d:], z], axis=0)
    return jnp.concatenate([z, v[:d]], axis=0)


def _mm(a, b, hi=True):
    prec = jax.lax.Precision.HIGHEST if hi else jax.lax.Precision.DEFAULT
    return jnp.dot(a, b, preferred_element_type=F32, precision=prec)


def _conv_s1(x, wt, b, hi=True):
    k = wt.shape[0]
    pad = (k - 1) // 2
    acc = jnp.broadcast_to(b, (x.shape[0], wt.shape[2])).astype(F32)
    for j in range(k):
        acc = acc + _mm(_shift(x, j - pad), wt[j], hi)
    return acc


def _conv_s2(xe, xo, wt, b, pad):
    k = wt.shape[0]
    acc = jnp.broadcast_to(b, (xe.shape[0], wt.shape[2])).astype(F32)
    for j in range(k):
        r = j - pad
        part = xe if (r % 2 == 0) else xo
        acc = acc + _mm(_shift(part, r // 2), wt[j])
    return acc


def _conv_t2(x, wt, b, hi=True):
    oe = b + _mm(x, wt[1], hi) + _mm(_shift(x, -1), wt[3], hi)
    oo = b + _mm(_shift(x, 1), wt[0], hi) + _mm(x, wt[2], hi)
    return oe, oo


def _resblock(x, w1, b1, w2, b2, hi=True):
    h = jax.nn.relu(x)
    h = _conv_s1(h, w1, b1, hi)
    h = jax.nn.relu(h)
    h = _mm(h, w2, hi) + b2
    return x + h


def _enc1_body(x_ref, w1t_ref, b1_ref, r0w1_ref, r0b1_ref, r0w2_ref, r0b2_ref,
               r1w1_ref, r1b1_ref, r1w2_ref, r1b2_ref, o_ref):
    xeo = x_ref[0]
    w1t = w1t_ref[...]
    b1 = b1_ref[...]
    xe = xeo[:, 0:1]
    xo = xeo[:, 1:2]
    acc = jnp.broadcast_to(b1, (xe.shape[0], 128)).astype(F32)
    for j in range(7):
        r = j - 3
        part = xe if (r % 2 == 0) else xo
        acc = acc + _shift(part, r // 2) * w1t[j][None, :]
    h = _resblock(acc, r0w1_ref[...], r0b1_ref[...], r0w2_ref[...], r0b2_ref[...])
    h = _resblock(h, r1w1_ref[...], r1b1_ref[...], r1w2_ref[...], r1b2_ref[...])
    o_ref[0] = h


def _enc2_body(xe_ref, xo_ref, wt_ref, b_ref, r0w1_ref, r0b1_ref, r0w2_ref,
               r0b2_ref, r1w1_ref, r1b1_ref, r1w2_ref, r1b2_ref, o_ref, *, pad):
    h = _conv_s2(xe_ref[0], xo_ref[0], wt_ref[...], b_ref[...], pad)
    h = _resblock(h, r0w1_ref[...], r0b1_ref[...], r0w2_ref[...], r0b2_ref[...])
    h = _resblock(h, r1w1_ref[...], r1b1_ref[...], r1w2_ref[...], r1b2_ref[...])
    o_ref[0] = h


def _enc3_body(xe_ref, xo_ref, wt_ref, b_ref, r0w1_ref, r0b1_ref, r0w2_ref,
               r0b2_ref, r1w1_ref, r1b1_ref, r1w2_ref, r1b2_ref,
               pw_ref, pb_ref, o_ref, *, pad):
    h = _conv_s2(xe_ref[0], xo_ref[0], wt_ref[...], b_ref[...], pad)
    h = _resblock(h, r0w1_ref[...], r0b1_ref[...], r0w2_ref[...], r0b2_ref[...])
    h = _resblock(h, r1w1_ref[...], r1b1_ref[...], r1w2_ref[...], r1b2_ref[...])
    o_ref[0] = _mm(h, pw_ref[...]) + pb_ref[...]


def _quantize_one(xf, embT, emb3):
    t = xf.shape[0]
    x2 = jnp.sum(xf * xf, axis=1, keepdims=True)
    best_d = jnp.full((t, 1), jnp.inf, F32)
    best_i = jnp.zeros((t, 1), jnp.int32)
    for c in range(N_CHUNKS):
        et = embT[:, c * CHUNK:(c + 1) * CHUNK]
        d = x2 - 2.0 * _mm(xf, et) + jnp.sum(et * et, axis=0, keepdims=True)
        m = jnp.min(d, axis=1, keepdims=True)
        iota = jax.lax.broadcasted_iota(jnp.int32, (t, CHUNK), 1)
        ii = jnp.min(jnp.where(d <= m, iota, K_CODES), axis=1, keepdims=True)
        upd = m < best_d
        best_d = jnp.where(upd, m, best_d)
        best_i = jnp.where(upd, ii + c * CHUNK, best_i)
    quant = jnp.zeros((t, D_CODE), F32)
    counts = []
    for c in range(N_CHUNKS):
        iota = jax.lax.broadcasted_iota(jnp.int32, (t, CHUNK), 1) + c * CHUNK
        oh = (best_i == iota).astype(F32)
        g = _mm(oh, emb3[0, c * CHUNK:(c + 1) * CHUNK], False)
        g = g + _mm(oh, emb3[1, c * CHUNK:(c + 1) * CHUNK], False)
        g = g + _mm(oh, emb3[2, c * CHUNK:(c + 1) * CHUNK], False)
        quant = quant + g
        counts.append(jnp.sum(oh, axis=0, keepdims=True))
    counts = jnp.concatenate(counts, axis=1)
    sumsq = jnp.sum((quant - xf) ** 2)
    return quant, counts, sumsq


def _quant_body(z_ref, e1t_ref, e1_ref, e2t_ref, e2_ref,
                q_ref, c1_ref, c2_ref, commit_ref, p1_ref, p2_ref, u_ref):
    b = pl.program_id(0)
    z = z_ref[0]
    q1, cnt1, s1 = _quantize_one(z[:, :D_CODE], e1t_ref[...], e1_ref[...])
    q2, cnt2, s2 = _quantize_one(z[:, D_CODE:], e2t_ref[...], e2_ref[...])
    q_ref[0] = jnp.concatenate([q1, q2], axis=1)

    @pl.when(b == 0)
    def _init():
        c1_ref[...] = cnt1
        c2_ref[...] = cnt2
        commit_ref[...] = jnp.full((1, 1), s1 + s2, F32)

    @pl.when(b > 0)
    def _acc():
        c1_ref[...] += cnt1
        c2_ref[...] += cnt2
        commit_ref[...] += s1 + s2

    @pl.when(b == NB - 1)
    def _finalize():
        n_rows = F32(NB * 1024)
        commit_ref[...] = commit_ref[...] * (CC / (NB * 1024 * D_CODE))
        for cref, pref in ((c1_ref, p1_ref), (c2_ref, p2_ref)):
            p = cref[...] / n_rows
            perp = jnp.exp(-jnp.sum(p * jnp.log(p + 1e-10)))
            pref[...] = jnp.full((1, 1), perp, F32)
        u1 = jnp.sum((c1_ref[...] / n_rows) *
                     jnp.log(c1_ref[...] / n_rows * K_CODES + 1e-10))
        u2 = jnp.sum((c2_ref[...] / n_rows) *
                     jnp.log(c2_ref[...] / n_rows * K_CODES + 1e-10))
        u_ref[...] = jnp.full((1, 1), u1 + u2, F32)


def _dec1_body(q_ref, pw_ref, pb_ref, r0w1_ref, r0b1_ref, r0w2_ref, r0b2_ref,
               r1w1_ref, r1b1_ref, r1w2_ref, r1b2_ref, uw_ref, ub_ref,
               oe_ref, oo_ref):
    h = _mm(q_ref[0], pw_ref[...], False) + pb_ref[...]
    h = _resblock(h, r0w1_ref[...], r0b1_ref[...], r0w2_ref[...], r0b2_ref[...],
                  False)
    h = _resblock(h, r1w1_ref[...], r1b1_ref[...], r1w2_ref[...], r1b2_ref[...],
                  False)
    oe, oo = _conv_t2(h, uw_ref[...], ub_ref[...], False)
    oe_ref[0] = oe
    oo_ref[0] = oo


def _dec23_body(y_ref, r0w1_ref, r0b1_ref, r0w2_ref, r0b2_ref,
                r1w1_ref, r1b1_ref, r1w2_ref, r1b2_ref, uw_ref, ub_ref,
                oe_ref, oo_ref):
    h = y_ref[0]
    h = _resblock(h, r0w1_ref[...], r0b1_ref[...], r0w2_ref[...], r0b2_ref[...],
                  False)
    h = _resblock(h, r1w1_ref[...], r1b1_ref[...], r1w2_ref[...], r1b2_ref[...],
                  False)
    oe, oo = _conv_t2(h, uw_ref[...], ub_ref[...], False)
    oe_ref[0] = oe
    oo_ref[0] = oo


def _out_body(y_ref, wt_ref, b_ref, o_ref):
    y = y_ref[0]
    wt = wt_ref[...]
    acc = jnp.zeros((y.shape[0], 1), F32)
    for j in range(7):
        acc = acc + jnp.sum(_shift(y, j - 3) * wt[j][None, :], axis=1,
                            keepdims=True)
    o_ref[0] = acc + b_ref[0, 0]


def _bspec(shape):
    nd = len(shape)
    return pl.BlockSpec((1,) + tuple(shape[1:]),
                        lambda b: (b,) + (0,) * (nd - 1))


def _wspec(shape):
    nd = len(shape)
    return pl.BlockSpec(tuple(shape), lambda b: (0,) * nd)


def _call(body, inputs, batch_in_mask, out_shapes, batch_out_mask):
    in_specs = [_bspec(a.shape) if m else _wspec(a.shape)
                for a, m in zip(inputs, batch_in_mask)]
    out_specs = [_bspec(s) if m else _wspec(s)
                 for s, m in zip(out_shapes, batch_out_mask)]
    out_shape = [jax.ShapeDtypeStruct(s, F32) for s in out_shapes]
    if len(out_shapes) == 1:
        out_specs = out_specs[0]
        out_shape = out_shape[0]
    return pl.pallas_call(
        body,
        grid=(NB,),
        in_specs=in_specs,
        out_specs=out_specs,
        out_shape=out_shape,
    )(*inputs)


def _kt(w):
    return jnp.transpose(w, (2, 1, 0))


def _ktT(w):
    return jnp.transpose(w, (2, 0, 1))


def _rb(p, prefix, i):
    return (_kt(p['%s_%d_w1' % (prefix, i)]),
            p['%s_%d_b1' % (prefix, i)][None, :],
            p['%s_%d_w2' % (prefix, i)][:, :, 0].T,
            p['%s_%d_b2' % (prefix, i)][None, :])


def kernel(x, params):
    p = params
    xeo = jnp.reshape(x[:, 0, :], (NB, 4096, 2))

    h1 = _call(
        _enc1_body,
        [xeo, jnp.transpose(p['conv1_w'][:, 0, :]), p['conv1_b'][None, :],
         *_rb(p, 'enc_res1', 0), *_rb(p, 'enc_res1', 1)],
        [True] + [False] * 10,
        [(NB, 4096, 128)], [True])

    h1r = jnp.reshape(h1, (NB, 2048, 2, 128))
    h2 = _call(
        functools.partial(_enc2_body, pad=2),
        [h1r[:, :, 0, :], h1r[:, :, 1, :], _kt(p['conv2_w']),
         p['conv2_b'][None, :], *_rb(p, 'enc_res2', 0), *_rb(p, 'enc_res2', 1)],
        [True, True] + [False] * 10,
        [(NB, 2048, 256)], [True])

    h2r = jnp.reshape(h2, (NB, 1024, 2, 256))
    z = _call(
        functools.partial(_enc3_body, pad=1),
        [h2r[:, :, 0, :], h2r[:, :, 1, :], _kt(p['conv3_w']),
         p['conv3_b'][None, :], *_rb(p, 'enc_res3', 0), *_rb(p, 'enc_res3', 1),
         p['enc_proj_w'][:, :, 0].T, p['enc_proj_b'][None, :]],
        [True, True] + [False] * 12,
        [(NB, 1024, 128)], [True])

    def _split3(e):
        hi = e.astype(jnp.bfloat16).astype(F32)
        r1 = e - hi
        mid = r1.astype(jnp.bfloat16).astype(F32)
        return jnp.stack([hi, mid, r1 - mid], axis=0)

    q, _c1, _c2, commit, perp1, perp2, usage = _call(
        _quant_body,
        [z, p['embed1'].T, _split3(p['embed1']), p['embed2'].T,
         _split3(p['embed2'])],
        [True] + [False] * 4,
        [(NB, 1024, 128), (1, K_CODES), (1, K_CODES), (1, 1), (1, 1), (1, 1),
         (1, 1)],
        [True, False, False, False, False, False, False])

    ye, yo = _call(
        _dec1_body,
        [q, p['dec_proj_w'][:, :, 0].T, p['dec_proj_b'][None, :],
         *_rb(p, 'dec_res1', 0), *_rb(p, 'dec_res1', 1),
         _ktT(p['up1_w']), p['up1_b'][None, :]],
        [True] + [False] * 12,
        [(NB, 1024, 256), (NB, 1024, 256)], [True, True])
    y = jnp.reshape(jnp.stack([ye, yo], axis=2), (NB, 2048, 256))

    ye, yo = _call(
        _dec23_body,
        [y, *_rb(p, 'dec_res2', 0), *_rb(p, 'dec_res2', 1),
         _ktT(p['up2_w']), p['up2_b'][None, :]],
        [True] + [False] * 10,
        [(NB, 2048, 128), (NB, 2048, 128)], [True, True])
    y = jnp.reshape(jnp.stack([ye, yo], axis=2), (NB, 4096, 128))

    ye, yo = _call(
        _dec23_body,
        [y, *_rb(p, 'dec_res3', 0), *_rb(p, 'dec_res3', 1),
         _ktT(p['up3_w']), p['up3_b'][None, :]],
        [True] + [False] * 10,
        [(NB, 4096, 128), (NB, 4096, 128)], [True, True])
    y = jnp.reshape(jnp.stack([ye, yo], axis=2), (NB, 8192, 128))

    recon = _call(
        _out_body,
        [y, jnp.transpose(p['out_w'][0]), p['out_b'][None, :]],
        [True, False, False],
        [(NB, 8192, 1)], [True])

    recon = jnp.transpose(recon, (0, 2, 1))
    return (recon, commit[0, 0], perp1[0, 0], perp2[0, 0], usage[0, 0])
